# baseline (speedup 1.0000x reference)
import functools
import jax
import jax.numpy as jnp
from jax import lax
from jax.experimental import pallas as pl
from jax.experimental.pallas import tpu as pltpu
from jax.experimental.pallas import tpu_sc as plsc

N = 10000
E = 320000
D = 128
PAD = 10240
NC = 2
NS = 16
EW = E // (NC * NS)
CH = 80
NT = EW // CH

_mesh = plsc.VectorSubcoreMesh(core_axis_name="c", subcore_axis_name="s")


@functools.partial(
    pl.kernel,
    out_type=jax.ShapeDtypeStruct((NC * PAD,), jnp.float32),
    mesh=_mesh,
    scratch_types=[
        pltpu.VMEM_SHARED((PAD,), jnp.float32),
        pltpu.VMEM((640,), jnp.float32),
        pltpu.VMEM((NT, CH), jnp.int32),
        pltpu.VMEM((CH,), jnp.float32),
    ],
)
def _deg_kernel(dst_hbm, out_hbm, acc, zbuf, idx, ones):
    c = lax.axis_index("c")
    s = lax.axis_index("s")
    zero16 = jnp.zeros((16,), jnp.float32)
    for i in range(40):
        zbuf[pl.ds(i * 16, 16)] = zero16
    one16 = jnp.full((16,), 1.0, jnp.float32)
    for i in range(CH // 16):
        ones[pl.ds(i * 16, 16)] = one16
    pltpu.sync_copy(zbuf, acc.at[pl.ds(s * 640, 640)])
    pltpu.sync_copy(dst_hbm.at[c * NS + s], idx)
    plsc.subcore_barrier()

    @pl.loop(0, NT)
    def _chunks(t):
        pltpu.sync_copy(ones, acc.at[idx.at[t]], add=True)

    plsc.subcore_barrier()
    pltpu.sync_copy(acc.at[pl.ds(s * 640, 640)],
                    out_hbm.at[pl.ds(c * PAD + s * 640, 640)])


@functools.partial(
    pl.kernel,
    out_type=jax.ShapeDtypeStruct((NC * PAD, D), jnp.float32),
    mesh=_mesh,
    scratch_types=[
        pltpu.VMEM_SHARED((PAD, D), jnp.float32),
        pltpu.VMEM((16, D), jnp.float32),
        pltpu.VMEM((NT, CH), jnp.int32),
        pltpu.VMEM((CH,), jnp.int32),
        pltpu.VMEM((CH,), jnp.int32),
        pltpu.VMEM((CH, D), jnp.float32),
        pltpu.VMEM((CH, D), jnp.float32),
        pltpu.SemaphoreType.DMA,
        pltpu.SemaphoreType.DMA,
        pltpu.SemaphoreType.DMA,
        pltpu.SemaphoreType.DMA,
    ],
)
def _agg_kernel(src_hbm, dst_hbm, g_hbm, out_hbm, acc, zbuf, di,
                s0, s1, r0, r1, i0, i1, g0, g1):
    c = lax.axis_index("c")
    s = lax.axis_index("s")
    zero16 = jnp.zeros((16,), jnp.float32)
    for i in range(16):
        for j in range(D // 16):
            zbuf[i, pl.ds(j * 16, 16)] = zero16

    @pl.loop(0, 40)
    def _zero(t):
        pltpu.sync_copy(zbuf, acc.at[pl.ds(s * 640 + t * 16, 16)])

    wid = c * NS + s
    base = wid * EW

    def sref(cix):
        return src_hbm.at[pl.ds(base + cix * CH, CH)]

    pltpu.sync_copy(dst_hbm.at[wid], di)
    plsc.subcore_barrier()

    pltpu.sync_copy(sref(0), s0)
    pltpu.async_copy(g_hbm.at[s0], r0, g0)
    pltpu.async_copy(sref(1), s1, i1)

    @pl.loop(0, (NT - 1) // 2)
    def _chunks(t):
        c0 = 2 * t
        pltpu.make_async_copy(sref(c0 + 1), s1, i1).wait()
        pltpu.async_copy(g_hbm.at[s1], r1, g1)
        pltpu.make_async_copy(g_hbm.at[s0], r0, g0).wait()
        pltpu.async_copy(sref(c0 + 2), s0, i0)
        pltpu.sync_copy(r0, acc.at[di.at[c0]], add=True)
        pltpu.make_async_copy(sref(c0 + 2), s0, i0).wait()
        pltpu.async_copy(g_hbm.at[s0], r0, g0)
        pltpu.make_async_copy(g_hbm.at[s1], r1, g1).wait()

        @pl.when(t < (NT - 1) // 2 - 1)
        def _pf():
            pltpu.async_copy(sref(c0 + 3), s1, i1)

        pltpu.sync_copy(r1, acc.at[di.at[c0 + 1]], add=True)

    pltpu.make_async_copy(g_hbm.at[s0], r0, g0).wait()
    pltpu.sync_copy(r0, acc.at[di.at[NT - 1]], add=True)
    plsc.subcore_barrier()

    @pl.loop(0, 5)
    def _out(t):
        r = s * 640 + t * 128
        pltpu.sync_copy(acc.at[pl.ds(r, 128)],
                        out_hbm.at[pl.ds(c * PAD + r, 128)])


def _dinv_of(dp):
    return lax.rsqrt(dp[0] + dp[1] + 1.0)[:N]


def _gemm_scale_body(dp_ref, x_ref, w_ref, o_ref):
    dv = _dinv_of(dp_ref[...])
    h = jnp.dot(x_ref[...], w_ref[...], preferred_element_type=jnp.float32)
    o_ref[...] = h * dv


def _layer2_body(dp_ref, sa_ref, sb_ref, g1_ref, w_ref, o_ref):
    dv = _dinv_of(dp_ref[...])
    z = jnp.maximum((sa_ref[...] + sb_ref[...] + g1_ref[...]) * dv, 0.0)
    h = jnp.dot(z, w_ref[...], preferred_element_type=jnp.float32)
    o_ref[...] = h * dv


def _final_body(dp_ref, sa_ref, sb_ref, g2_ref, o_ref):
    dv = _dinv_of(dp_ref[...])
    o_ref[...] = (sa_ref[...] + sb_ref[...] + g2_ref[...]) * dv


def kernel(args, x, edge_index, W1, W2):
    src1d = edge_index[0]
    dst2d = edge_index[1].reshape(NC * NS, NT, CH)

    dp = _deg_kernel(dst2d).reshape(NC, PAD, 1)

    g1 = pl.pallas_call(
        _gemm_scale_body,
        out_shape=jax.ShapeDtypeStruct((N, D), jnp.float32))(dp, x, W1)

    s1 = _agg_kernel(src1d, dst2d, g1)
    g2 = pl.pallas_call(
        _layer2_body,
        out_shape=jax.ShapeDtypeStruct((N, D), jnp.float32))(
            dp, s1[:N], s1[PAD:PAD + N], g1, W2)

    s2 = _agg_kernel(src1d, dst2d, g2)
    out = pl.pallas_call(
        _final_body,
        out_shape=jax.ShapeDtypeStruct((N, D), jnp.float32))(
            dp, s2[:N], s2[PAD:PAD + N], g2)
    return out


# speedup vs baseline: 1.0688x; 1.0688x over previous
import functools
import jax
import jax.numpy as jnp
from jax import lax
from jax.experimental import pallas as pl
from jax.experimental.pallas import tpu as pltpu
from jax.experimental.pallas import tpu_sc as plsc

N = 10000
E = 320000
D = 128
PAD = 10240
NC = 2
NS = 16
EW = E // (NC * NS)
CH = 80
NT = EW // CH

_mesh = plsc.VectorSubcoreMesh(core_axis_name="c", subcore_axis_name="s")


@functools.partial(
    pl.kernel,
    out_type=jax.ShapeDtypeStruct((NC * PAD,), jnp.float32),
    mesh=_mesh,
    scratch_types=[
        pltpu.VMEM_SHARED((PAD,), jnp.float32),
        pltpu.VMEM((640,), jnp.float32),
        pltpu.VMEM((NT, CH), jnp.int32),
        pltpu.VMEM((CH,), jnp.float32),
    ],
)
def _deg_kernel(dst_hbm, out_hbm, acc, zbuf, idx, ones):
    c = lax.axis_index("c")
    s = lax.axis_index("s")
    zero16 = jnp.zeros((16,), jnp.float32)
    for i in range(40):
        zbuf[pl.ds(i * 16, 16)] = zero16
    one16 = jnp.full((16,), 1.0, jnp.float32)
    for i in range(CH // 16):
        ones[pl.ds(i * 16, 16)] = one16
    pltpu.sync_copy(zbuf, acc.at[pl.ds(s * 640, 640)])
    pltpu.sync_copy(dst_hbm.at[c * NS + s], idx)
    plsc.subcore_barrier()

    @pl.loop(0, NT)
    def _chunks(t):
        pltpu.sync_copy(ones, acc.at[idx.at[t]], add=True)

    plsc.subcore_barrier()
    pltpu.sync_copy(acc.at[pl.ds(s * 640, 640)],
                    out_hbm.at[pl.ds(c * PAD + s * 640, 640)])


@functools.partial(
    pl.kernel,
    out_type=jax.ShapeDtypeStruct((NC * PAD, D), jnp.float32),
    mesh=_mesh,
    scratch_types=[
        pltpu.VMEM_SHARED((PAD, D), jnp.float32),
        pltpu.VMEM((16, D), jnp.float32),
        pltpu.VMEM((NT, CH), jnp.int32),
        pltpu.VMEM((CH,), jnp.int32),
        pltpu.VMEM((CH,), jnp.int32),
        pltpu.VMEM((CH, D), jnp.float32),
        pltpu.VMEM((CH, D), jnp.float32),
        pltpu.SemaphoreType.DMA,
        pltpu.SemaphoreType.DMA,
        pltpu.SemaphoreType.DMA,
        pltpu.SemaphoreType.DMA,
    ],
)
def _agg_kernel(src_hbm, dst_hbm, g_hbm, out_hbm, acc, zbuf, di,
                s0, s1, r0, r1, i0, i1, g0, g1):
    c = lax.axis_index("c")
    s = lax.axis_index("s")
    zero16 = jnp.zeros((16,), jnp.float32)
    for i in range(16):
        for j in range(D // 16):
            zbuf[i, pl.ds(j * 16, 16)] = zero16

    @pl.loop(0, 40)
    def _zero(t):
        pltpu.sync_copy(zbuf, acc.at[pl.ds(s * 640 + t * 16, 16)])

    wid = c * NS + s
    base = wid * EW

    def sref(cix):
        return src_hbm.at[pl.ds(base + cix * CH, CH)]

    pltpu.sync_copy(dst_hbm.at[wid], di)
    plsc.subcore_barrier()

    pltpu.sync_copy(sref(0), s0)
    pltpu.async_copy(g_hbm.at[s0], r0, g0)
    pltpu.async_copy(sref(1), s1, i1)

    @pl.loop(0, (NT - 1) // 2)
    def _chunks(t):
        c0 = 2 * t
        pltpu.make_async_copy(sref(c0 + 1), s1, i1).wait()
        pltpu.async_copy(g_hbm.at[s1], r1, g1)
        pltpu.make_async_copy(g_hbm.at[s0], r0, g0).wait()
        pltpu.async_copy(sref(c0 + 2), s0, i0)
        pltpu.sync_copy(r0, acc.at[di.at[c0]], add=True)
        pltpu.make_async_copy(sref(c0 + 2), s0, i0).wait()
        pltpu.async_copy(g_hbm.at[s0], r0, g0)
        pltpu.make_async_copy(g_hbm.at[s1], r1, g1).wait()

        @pl.when(t < (NT - 1) // 2 - 1)
        def _pf():
            pltpu.async_copy(sref(c0 + 3), s1, i1)

        pltpu.sync_copy(r1, acc.at[di.at[c0 + 1]], add=True)

    pltpu.make_async_copy(g_hbm.at[s0], r0, g0).wait()
    pltpu.sync_copy(r0, acc.at[di.at[NT - 1]], add=True)
    plsc.subcore_barrier()

    @pl.loop(0, 5)
    def _out(t):
        r = s * 640 + t * 128
        pltpu.sync_copy(acc.at[pl.ds(r, 128)],
                        out_hbm.at[pl.ds(c * PAD + r, 128)])


def _dinv_body(dp_ref, o_ref):
    d = dp_ref[...]
    deg = d[:PAD] + d[PAD:] + 1.0
    o_ref[...] = lax.rsqrt(deg)


def _gemm_scale_body(x_ref, w_ref, dv_ref, o_ref):
    h = jnp.dot(x_ref[...], w_ref[...], preferred_element_type=jnp.float32)
    o_ref[...] = h * dv_ref[...]


def _layer2_body(s_ref, g1_ref, dv_ref, w_ref, o_ref):
    dv = dv_ref[...]
    sboth = s_ref[...]
    ssum = sboth[:N] + sboth[PAD:PAD + N] + g1_ref[...]
    z = jnp.maximum(ssum * dv, 0.0)
    h = jnp.dot(z, w_ref[...], preferred_element_type=jnp.float32)
    o_ref[...] = h * dv


def _final_body(s_ref, g2_ref, dv_ref, o_ref):
    sboth = s_ref[...]
    o_ref[...] = (sboth[:N] + sboth[PAD:PAD + N] + g2_ref[...]) * dv_ref[...]


def kernel(args, x, edge_index, W1, W2):
    src1d = edge_index[0]
    dst2d = edge_index[1].reshape(NC * NS, NT, CH)

    dp = _deg_kernel(dst2d)
    dinv = pl.pallas_call(
        _dinv_body, out_shape=jax.ShapeDtypeStruct((PAD,), jnp.float32))(dp)
    dcol = dinv[:N].reshape(N, 1)

    g1 = pl.pallas_call(
        _gemm_scale_body,
        out_shape=jax.ShapeDtypeStruct((N, D), jnp.float32))(x, W1, dcol)

    s1 = _agg_kernel(src1d, dst2d, g1)
    g2 = pl.pallas_call(
        _layer2_body,
        out_shape=jax.ShapeDtypeStruct((N, D), jnp.float32))(s1, g1, dcol, W2)

    s2 = _agg_kernel(src1d, dst2d, g2)
    out = pl.pallas_call(
        _final_body,
        out_shape=jax.ShapeDtypeStruct((N, D), jnp.float32))(s2, g2, dcol)
    return out


# speedup vs baseline: 1.1150x; 1.0432x over previous
import functools
import jax
import jax.numpy as jnp
from jax import lax
from jax.experimental import pallas as pl
from jax.experimental.pallas import tpu as pltpu
from jax.experimental.pallas import tpu_sc as plsc

N = 10000
E = 320000
D = 128
PAD = 10240
NC = 2
NS = 16
EW = E // (NC * NS)
CH = 80
NT = EW // CH

_mesh = plsc.VectorSubcoreMesh(core_axis_name="c", subcore_axis_name="s")


@functools.partial(
    pl.kernel,
    out_type=jax.ShapeDtypeStruct((NC * PAD,), jnp.float32),
    mesh=_mesh,
    scratch_types=[
        pltpu.VMEM_SHARED((PAD,), jnp.float32),
        pltpu.VMEM((640,), jnp.float32),
        pltpu.VMEM((NT, CH), jnp.int32),
        pltpu.VMEM((CH,), jnp.float32),
        pltpu.SemaphoreType.DMA,
    ],
)
def _deg_kernel(dst_hbm, out_hbm, acc, zbuf, idx, ones, dsem):
    c = lax.axis_index("c")
    s = lax.axis_index("s")
    zero16 = jnp.zeros((16,), jnp.float32)
    for i in range(40):
        zbuf[pl.ds(i * 16, 16)] = zero16
    one16 = jnp.full((16,), 1.0, jnp.float32)
    for i in range(CH // 16):
        ones[pl.ds(i * 16, 16)] = one16
    pltpu.sync_copy(zbuf, acc.at[pl.ds(s * 640, 640)])
    pltpu.sync_copy(dst_hbm.at[c * NS + s], idx)
    plsc.subcore_barrier()

    @pl.loop(0, NT)
    def _chunks(t):
        pltpu.async_copy(ones, acc.at[idx.at[t]], dsem, add=True)

        @pl.when(t >= 8)
        def _drain():
            pltpu.make_async_copy(ones, acc.at[idx.at[0]], dsem).wait()

    @pl.loop(0, 8)
    def _drain_tail(t):
        pltpu.make_async_copy(ones, acc.at[idx.at[0]], dsem).wait()

    plsc.subcore_barrier()
    pltpu.sync_copy(acc.at[pl.ds(s * 640, 640)],
                    out_hbm.at[pl.ds(c * PAD + s * 640, 640)])


@functools.partial(
    pl.kernel,
    out_type=jax.ShapeDtypeStruct((NC * PAD, D), jnp.float32),
    mesh=_mesh,
    scratch_types=[
        pltpu.VMEM_SHARED((PAD, D), jnp.float32),
        pltpu.VMEM((40, D), jnp.float32),
        pltpu.VMEM((NT, CH), jnp.int32),
        pltpu.VMEM((CH,), jnp.int32),
        pltpu.VMEM((CH,), jnp.int32),
        pltpu.VMEM((CH, D), jnp.float32),
        pltpu.VMEM((CH, D), jnp.float32),
        pltpu.SemaphoreType.DMA,
        pltpu.SemaphoreType.DMA,
        pltpu.SemaphoreType.DMA,
        pltpu.SemaphoreType.DMA,
    ],
)
def _agg_kernel(src_hbm, dst_hbm, g_hbm, out_hbm, acc, zbuf, di,
                s0, s1, r0, r1, i0, i1, g0, g1):
    c = lax.axis_index("c")
    s = lax.axis_index("s")
    zero16 = jnp.zeros((16,), jnp.float32)
    for i in range(40):
        for j in range(D // 16):
            zbuf[i, pl.ds(j * 16, 16)] = zero16

    @pl.loop(0, 16)
    def _zero(t):
        pltpu.async_copy(zbuf, acc.at[pl.ds(s * 640 + t * 40, 40)], g0)

    wid = c * NS + s
    base = wid * EW

    def sref(cix):
        return src_hbm.at[pl.ds(base + cix * CH, CH)]

    pltpu.sync_copy(dst_hbm.at[wid], di)

    @pl.loop(0, 16)
    def _zero_drain(t):
        pltpu.make_async_copy(zbuf, acc.at[pl.ds(s * 640, 40)], g0).wait()

    plsc.subcore_barrier()

    pltpu.sync_copy(sref(0), s0)
    pltpu.async_copy(g_hbm.at[s0], r0, g0)
    pltpu.async_copy(sref(1), s1, i1)

    @pl.loop(0, (NT - 1) // 2)
    def _chunks(t):
        c0 = 2 * t
        pltpu.make_async_copy(sref(c0 + 1), s1, i1).wait()
        pltpu.async_copy(g_hbm.at[s1], r1, g1)
        pltpu.make_async_copy(g_hbm.at[s0], r0, g0).wait()
        pltpu.async_copy(sref(c0 + 2), s0, i0)
        pltpu.sync_copy(r0, acc.at[di.at[c0]], add=True)
        pltpu.make_async_copy(sref(c0 + 2), s0, i0).wait()
        pltpu.async_copy(g_hbm.at[s0], r0, g0)
        pltpu.make_async_copy(g_hbm.at[s1], r1, g1).wait()

        @pl.when(t < (NT - 1) // 2 - 1)
        def _pf():
            pltpu.async_copy(sref(c0 + 3), s1, i1)

        pltpu.sync_copy(r1, acc.at[di.at[c0 + 1]], add=True)

    pltpu.make_async_copy(g_hbm.at[s0], r0, g0).wait()
    pltpu.sync_copy(r0, acc.at[di.at[NT - 1]], add=True)
    plsc.subcore_barrier()

    @pl.loop(0, 5)
    def _out(t):
        r = s * 640 + t * 128
        pltpu.sync_copy(acc.at[pl.ds(r, 128)],
                        out_hbm.at[pl.ds(c * PAD + r, 128)])


def _dinv_body(dp_ref, o_ref):
    d = dp_ref[...]
    deg = d[:PAD] + d[PAD:] + 1.0
    o_ref[...] = lax.rsqrt(deg)


def _gemm_scale_body(x_ref, w_ref, dv_ref, o_ref):
    h = jnp.dot(x_ref[...], w_ref[...], preferred_element_type=jnp.float32)
    o_ref[...] = h * dv_ref[...]


def _layer2_body(s_ref, g1_ref, dv_ref, w_ref, o_ref):
    dv = dv_ref[...]
    sboth = s_ref[...]
    ssum = sboth[:N] + sboth[PAD:PAD + N] + g1_ref[...]
    z = jnp.maximum(ssum * dv, 0.0)
    h = jnp.dot(z, w_ref[...], preferred_element_type=jnp.float32)
    o_ref[...] = h * dv


def _final_body(s_ref, g2_ref, dv_ref, o_ref):
    sboth = s_ref[...]
    o_ref[...] = (sboth[:N] + sboth[PAD:PAD + N] + g2_ref[...]) * dv_ref[...]


def kernel(args, x, edge_index, W1, W2):
    src1d = edge_index[0]
    dst2d = edge_index[1].reshape(NC * NS, NT, CH)

    dp = _deg_kernel(dst2d)
    dinv = pl.pallas_call(
        _dinv_body, out_shape=jax.ShapeDtypeStruct((PAD,), jnp.float32))(dp)
    dcol = dinv[:N].reshape(N, 1)

    g1 = pl.pallas_call(
        _gemm_scale_body,
        out_shape=jax.ShapeDtypeStruct((N, D), jnp.float32))(x, W1, dcol)

    s1 = _agg_kernel(src1d, dst2d, g1)
    g2 = pl.pallas_call(
        _layer2_body,
        out_shape=jax.ShapeDtypeStruct((N, D), jnp.float32))(s1, g1, dcol, W2)

    s2 = _agg_kernel(src1d, dst2d, g2)
    out = pl.pallas_call(
        _final_body,
        out_shape=jax.ShapeDtypeStruct((N, D), jnp.float32))(s2, g2, dcol)
    return out
